# baseline (speedup 1.0000x reference)
import jax
import jax.numpy as jnp
from jax import lax
from jax.experimental import pallas as pl
from jax.experimental.pallas import tpu as pltpu
from jax.experimental.pallas import tpu_sc as plsc

VOCAB = 1000000
EMB = 24
W = 128


def _sc_body(idx_hbm, mt_hbm, uv_hbm, out_hbm, idx_v, uv_v, blk_v, res_v, sem):
    cid = lax.axis_index("c")
    sid = lax.axis_index("s")

    @pl.when((cid == 0) & (sid == 0))
    def _():
        cp_i = pltpu.make_async_copy(idx_hbm, idx_v.at[pl.ds(0, 1)], sem)
        cp_i.start()
        cp_u = pltpu.make_async_copy(uv_hbm, uv_v, sem)
        cp_u.start()
        cp_i.wait()
        cp_u.wait()
        i = idx_v[...][0]
        base = pl.multiple_of((i // W) * W, W)
        pltpu.sync_copy(mt_hbm.at[:, pl.ds(base, W)], blk_v)
        lane = i - base
        c16 = lax.iota(jnp.int32, 16)
        lane_v = jnp.full((16,), lane, jnp.int32)
        g1 = plsc.load_gather(blk_v, [c16, lane_v])
        u1 = plsc.load_gather(uv_v, [c16])
        c2 = c16 + 16
        c2c = jnp.minimum(c2, EMB - 1)
        g2 = plsc.load_gather(blk_v, [c2c, lane_v])
        u2 = plsc.load_gather(uv_v, [c2c])
        valid = c2 < EMB
        p = g1 * u1 + jnp.where(valid, g2 * u2, 0.0)
        s = jnp.sum(p)
        x = jnp.full((16,), s, jnp.float32)
        sig = 1.0 / (1.0 + jnp.exp(-x))
        res_v[...] = sig
        pltpu.sync_copy(res_v.at[pl.ds(0, 1)], out_hbm)


def kernel(input, matrix, user_vector):
    idx = jnp.asarray(input, jnp.int32).reshape((1,))
    mt = matrix.T
    uv = user_vector.reshape((EMB,))
    mesh = plsc.VectorSubcoreMesh(core_axis_name="c", subcore_axis_name="s")
    k = pl.kernel(
        _sc_body,
        out_type=jax.ShapeDtypeStruct((1,), jnp.float32),
        mesh=mesh,
        scratch_types=[
            pltpu.VMEM((16,), jnp.int32),
            pltpu.VMEM((EMB,), jnp.float32),
            pltpu.VMEM((EMB, W), jnp.float32),
            pltpu.VMEM((16,), jnp.float32),
            pltpu.SemaphoreType.DMA,
        ],
        compiler_params=pltpu.CompilerParams(
            use_tc_tiling_on_sc=True, needs_layout_passes=False),
    )
    return k(idx, mt, uv)


# speedup vs baseline: 1.0257x; 1.0257x over previous
import jax
import jax.numpy as jnp
from jax import lax
from jax.experimental import pallas as pl
from jax.experimental.pallas import tpu as pltpu
from jax.experimental.pallas import tpu_sc as plsc

VOCAB = 1000000
EMB = 24
W = 128


def _sc_body(idx_hbm, mt_hbm, uv_hbm, out_hbm, idx_v, uv_v, blk_v, res_v, sem):
    cid = lax.axis_index("c")
    sid = lax.axis_index("s")

    @pl.when((cid == 0) & (sid == 0))
    def _():
        cp_i = pltpu.make_async_copy(idx_hbm, idx_v.at[pl.ds(0, 1)], sem)
        cp_i.start()
        cp_u = pltpu.make_async_copy(uv_hbm, uv_v, sem)
        cp_u.start()
        cp_i.wait()
        cp_u.wait()
        i = idx_v[...][0]
        base = pl.multiple_of((i // W) * W, W)
        pltpu.sync_copy(mt_hbm.at[:, pl.ds(base, W)], blk_v)
        lane = i - base
        c16 = lax.iota(jnp.int32, 16)
        lane_v = jnp.full((16,), lane, jnp.int32)
        g1 = plsc.load_gather(blk_v, [c16, lane_v])
        u1 = plsc.load_gather(uv_v, [c16])
        c2 = c16 + 16
        c2c = jnp.minimum(c2, EMB - 1)
        g2 = plsc.load_gather(blk_v, [c2c, lane_v])
        u2 = plsc.load_gather(uv_v, [c2c])
        valid = c2 < EMB
        p = g1 * u1 + jnp.where(valid, g2 * u2, 0.0)
        s = jnp.sum(p)
        x = jnp.full((16,), s, jnp.float32)
        sig = 1.0 / (1.0 + jnp.exp(-x))
        res_v[...] = sig
        pltpu.sync_copy(res_v.at[pl.ds(0, 1)], out_hbm)


def kernel(input, matrix, user_vector):
    idx = jnp.asarray(input, jnp.int32).reshape((1,))
    mt = matrix.T
    uv = user_vector.reshape((EMB,))
    mesh = plsc.VectorSubcoreMesh(core_axis_name="c", subcore_axis_name="s")
    k = pl.kernel(
        _sc_body,
        out_type=jax.ShapeDtypeStruct((1,), jnp.float32),
        mesh=mesh,
        scratch_types=[
            pltpu.VMEM((16,), jnp.int32),
            pltpu.VMEM((EMB,), jnp.float32),
            pltpu.VMEM((EMB, W), jnp.float32),
            pltpu.VMEM((16,), jnp.float32),
            pltpu.SemaphoreType.DMA,
        ],
        compiler_params=pltpu.CompilerParams(
            use_tc_tiling_on_sc=True, needs_layout_passes=False,
            skip_device_barrier=True),
    )
    return k(idx, mt, uv)


# speedup vs baseline: 7.3930x; 7.2075x over previous
import jax
import jax.numpy as jnp
from jax.experimental import pallas as pl
from jax.experimental.pallas import tpu as pltpu

VOCAB = 1000000
EMB = 24
W = 128


def _lookup_kernel(idx_ref, hbm_ref, uv_ref, out_ref, blk_vmem, sem):
    i = idx_ref[0]
    base = pl.multiple_of((i // W) * W, W)
    cp = pltpu.make_async_copy(hbm_ref.at[:, pl.ds(base, W)], blk_vmem, sem)
    cp.start()
    cp.wait()
    prods = jnp.dot(uv_ref[...], blk_vmem[...],
                    preferred_element_type=jnp.float32)
    lane = i - base
    mask = jax.lax.broadcasted_iota(jnp.int32, (1, W), 1) == lane
    s = jnp.sum(jnp.where(mask, prods, 0.0), keepdims=True).reshape(1, 1)
    out_ref[...] = jax.nn.sigmoid(s)


def kernel(input, matrix, user_vector):
    idx = jnp.asarray(input, jnp.int32).reshape((1,))
    mt = matrix.T
    out = pl.pallas_call(
        _lookup_kernel,
        in_specs=[
            pl.BlockSpec(memory_space=pltpu.SMEM),
            pl.BlockSpec(memory_space=pl.ANY),
            pl.BlockSpec(memory_space=pltpu.VMEM),
        ],
        out_specs=pl.BlockSpec(memory_space=pltpu.VMEM),
        out_shape=jax.ShapeDtypeStruct((1, 1), jnp.float32),
        scratch_shapes=[
            pltpu.VMEM((EMB, W), jnp.float32),
            pltpu.SemaphoreType.DMA,
        ],
    )(idx, mt, user_vector)
    return out.reshape((1,))


# speedup vs baseline: 7.4353x; 1.0057x over previous
import jax
import jax.numpy as jnp
from jax.experimental import pallas as pl
from jax.experimental.pallas import tpu as pltpu

VOCAB = 1000000
EMB = 24
W = 128


def _lookup_kernel(idx_ref, hbm_ref, uv_ref, out_ref, blk_vmem, sem):
    i = idx_ref[()]
    base = pl.multiple_of((i // W) * W, W)
    cp = pltpu.make_async_copy(hbm_ref.at[:, pl.ds(base, W)], blk_vmem, sem)
    cp.start()
    cp.wait()
    prods = jnp.dot(uv_ref[...], blk_vmem[...],
                    preferred_element_type=jnp.float32)
    lane = i - base
    mask = jax.lax.broadcasted_iota(jnp.int32, (1, W), 1) == lane
    s = jnp.sum(jnp.where(mask, prods, 0.0), keepdims=True).reshape(1, 1)
    out_ref[...] = jax.nn.sigmoid(s)


def kernel(input, matrix, user_vector):
    idx = jnp.asarray(input, jnp.int32)
    mt = matrix.T
    out = pl.pallas_call(
        _lookup_kernel,
        in_specs=[
            pl.BlockSpec(memory_space=pltpu.SMEM),
            pl.BlockSpec(memory_space=pl.ANY),
            pl.BlockSpec(memory_space=pltpu.VMEM),
        ],
        out_specs=pl.BlockSpec(memory_space=pltpu.VMEM),
        out_shape=jax.ShapeDtypeStruct((1, 1), jnp.float32),
        scratch_shapes=[
            pltpu.VMEM((EMB, W), jnp.float32),
            pltpu.SemaphoreType.DMA,
        ],
    )(idx, mt, user_vector)
    return out.reshape((1,))


# speedup vs baseline: 7.4718x; 1.0049x over previous
import jax
import jax.numpy as jnp
from jax.experimental import pallas as pl
from jax.experimental.pallas import tpu as pltpu

VOCAB = 1000000
EMB = 24
W = 128


def _lookup_kernel(idx_ref, hbm_ref, uv_ref, out_ref, blk_vmem, sem):
    i = idx_ref[()]
    base = pl.multiple_of((i // W) * W, W)
    cp = pltpu.make_async_copy(hbm_ref.at[:, pl.ds(base, W)], blk_vmem, sem)
    cp.start()
    cp.wait()
    prods = jnp.dot(uv_ref[...], blk_vmem[...],
                    preferred_element_type=jnp.float32,
                    precision=jax.lax.Precision.HIGHEST)
    lane = i - base
    mask = jax.lax.broadcasted_iota(jnp.int32, (1, W), 1) == lane
    s = jnp.sum(jnp.where(mask, prods, 0.0), keepdims=True).reshape(1, 1)
    out_ref[...] = jax.nn.sigmoid(s)


def kernel(input, matrix, user_vector):
    idx = jnp.asarray(input, jnp.int32)
    mt = matrix.T
    out = pl.pallas_call(
        _lookup_kernel,
        in_specs=[
            pl.BlockSpec(memory_space=pltpu.SMEM),
            pl.BlockSpec(memory_space=pl.ANY),
            pl.BlockSpec(memory_space=pltpu.VMEM),
        ],
        out_specs=pl.BlockSpec(memory_space=pltpu.VMEM),
        out_shape=jax.ShapeDtypeStruct((1, 1), jnp.float32),
        scratch_shapes=[
            pltpu.VMEM((EMB, W), jnp.float32),
            pltpu.SemaphoreType.DMA,
        ],
    )(idx, mt, user_vector)
    return out.reshape((1,))
